# Initial kernel scaffold; baseline (speedup 1.0000x reference)
import jax
import jax.numpy as jnp
from jax.experimental import pallas as pl


def kernel(x, edge_index, W1, b1, gamma, beta, W2, b2):
    raise NotImplementedError("write your pallas kernel here")


# speedup vs baseline: 5.6534x; 5.6534x over previous
import functools

import jax
import jax.numpy as jnp
from jax import lax
from jax.experimental import pallas as pl
from jax.experimental.pallas import tpu as pltpu
from jax.experimental.pallas import tpu_sc as plsc

N = 10000
E = 160000
D_IN = 256
D_H = 512
NCH = 4
CW = 128
NPAD = 10112
NB = 10240
NC, NS = 2, 16
NBATCH = 80
K = 128
EP = NS * NBATCH * K
DEG_PER_W = 5008
EPD = NC * NS * DEG_PER_W
MBLK = 400
GRID_M = N // MBLK
BN_SCALE = (1.0 + 1e-5) ** -0.5

_mesh = plsc.VectorSubcoreMesh(core_axis_name="c", subcore_axis_name="s")
_sc_params = pltpu.CompilerParams(needs_layout_passes=False)


def _deg_body(dst_hbm, hist_hbm, idx_v, hist_v):
    c = lax.axis_index("c")
    s = lax.axis_index("s")
    wid = c * NS + s
    pltpu.sync_copy(dst_hbm.at[wid], idx_v)
    zeros16 = jnp.zeros((16,), jnp.float32)
    ones16 = jnp.ones((16,), jnp.float32)

    def zbody(i, carry):
        hist_v[pl.ds(i * 16, 16)] = zeros16
        return carry

    lax.fori_loop(0, NB // 16, zbody, 0)

    def hbody(j, carry):
        idx = idx_v[pl.ds(j * 16, 16)]
        plsc.addupdate_scatter(hist_v, [idx], ones16)
        return carry

    lax.fori_loop(0, DEG_PER_W // 16, hbody, 0)
    pltpu.sync_copy(hist_v, hist_hbm.at[wid])


_deg_call = functools.partial(
    pl.kernel,
    out_type=jax.ShapeDtypeStruct((NC * NS, NB), jnp.float32),
    mesh=_mesh,
    scratch_types=[
        pltpu.VMEM((DEG_PER_W,), jnp.int32),
        pltpu.VMEM((NB,), jnp.float32),
    ],
    compiler_params=_sc_params,
)(_deg_body)


def _scatter_body(y_hbm, src_hbm, dst_hbm, z_hbm,
                  src_v, dst_v, rowbuf, acc, sem):
    c = lax.axis_index("c")
    s = lax.axis_index("s")
    pltpu.sync_copy(src_hbm.at[s], src_v)
    pltpu.sync_copy(dst_hbm.at[s], dst_v)

    zeros16 = jnp.zeros((16,), jnp.float32)

    own = [(0, 128), (128, 128), (256, 128), (384, 128), (512, 120)]
    base = s * (NPAD // NS)

    def process_chunk(chunk):
        y_view = y_hbm.at[chunk]
        z_view = z_hbm.at[chunk]

        def zb(r, carry):
            for k in range(CW // 16):
                rowbuf[r, pl.ds(k * 16, 16)] = zeros16
            return carry

        lax.fori_loop(0, K, zb, 0)
        for r0, sz in own:
            pltpu.sync_copy(rowbuf.at[pl.ds(0, sz)],
                            acc.at[pl.ds(base + r0, sz)])
        plsc.subcore_barrier()

        def ebody(j, carry):
            pltpu.async_copy(y_view.at[src_v.at[j]], rowbuf, sem).wait()
            pltpu.sync_copy(rowbuf, acc.at[dst_v.at[j]], add=True)
            return carry

        lax.fori_loop(0, NBATCH, ebody, 0)
        plsc.subcore_barrier()
        for r0, sz in own:
            pltpu.sync_copy(acc.at[pl.ds(base + r0, sz)],
                            z_view.at[pl.ds(base + r0, sz)])

    for cc in range(NCH // NC):
        for core in range(NC):
            @pl.when(c == core)
            def _(core=core, cc=cc):
                process_chunk(core * (NCH // NC) + cc)


_scatter_call = functools.partial(
    pl.kernel,
    out_type=jax.ShapeDtypeStruct((NCH, NPAD, CW), jnp.float32),
    mesh=_mesh,
    scratch_types=[
        pltpu.VMEM((NBATCH, K), jnp.int32),
        pltpu.VMEM((NBATCH, K), jnp.int32),
        pltpu.VMEM((K, CW), jnp.float32),
        pltpu.VMEM_SHARED((NPAD, CW), jnp.float32),
        pltpu.SemaphoreType.DMA,
    ],
    compiler_params=_sc_params,
)(_scatter_body)


def _dis_body(hist_ref, out_ref):
    h = hist_ref[...]
    deg = jnp.sum(h, axis=0, keepdims=True) + 1.0
    d = lax.rsqrt(deg)
    dcol = lax.transpose(d, (1, 0))
    out_ref[...] = jnp.broadcast_to(dcol[:N], (N, CW))


def _dis_call(hist):
    return pl.pallas_call(
        _dis_body,
        out_shape=jax.ShapeDtypeStruct((N, CW), jnp.float32),
    )(hist)


def _mm1_body(x_ref, w_ref, dis_ref, out_ref):
    xb = x_ref[...]
    w = w_ref[...]
    dis = dis_ref[...]
    for ci in range(NCH):
        out_ref[ci] = dis * jnp.dot(xb, w[:, ci * CW:(ci + 1) * CW],
                                    preferred_element_type=jnp.float32)


def _mm1_call(x, W1, dis_b):
    return pl.pallas_call(
        _mm1_body,
        grid=(GRID_M,),
        in_specs=[
            pl.BlockSpec((MBLK, D_IN), lambda i: (i, 0)),
            pl.BlockSpec((D_IN, D_H), lambda i: (0, 0)),
            pl.BlockSpec((MBLK, CW), lambda i: (i, 0)),
        ],
        out_specs=pl.BlockSpec((NCH, MBLK, CW), lambda i: (0, i, 0)),
        out_shape=jax.ShapeDtypeStruct((NCH, N, CW), jnp.float32),
    )(x, W1, dis_b)


def _mid_body(z_ref, y_ref, dis_ref, b1_ref, g_ref, bt_ref, w2_ref, out_ref):
    dis = dis_ref[...]
    w2 = w2_ref[...]
    acc = jnp.zeros((MBLK, D_H), jnp.float32)
    for ci in range(NCH):
        t = dis * (z_ref[ci] + y_ref[ci]) + b1_ref[ci]
        t = t * (BN_SCALE * g_ref[ci]) + bt_ref[ci]
        t = jnp.maximum(t, 0.0)
        acc = acc + jnp.dot(t, w2[ci * CW:(ci + 1) * CW, :],
                            preferred_element_type=jnp.float32)
    y2 = dis[:, 0:1] * acc
    for co in range(NCH):
        out_ref[co] = y2[:, co * CW:(co + 1) * CW]


def _mid_call(z1, y1, dis_b, b1c, gc, btc, W2):
    return pl.pallas_call(
        _mid_body,
        grid=(GRID_M,),
        in_specs=[
            pl.BlockSpec((NCH, MBLK, CW), lambda i: (0, i, 0)),
            pl.BlockSpec((NCH, MBLK, CW), lambda i: (0, i, 0)),
            pl.BlockSpec((MBLK, CW), lambda i: (i, 0)),
            pl.BlockSpec((NCH, 1, CW), lambda i: (0, 0, 0)),
            pl.BlockSpec((NCH, 1, CW), lambda i: (0, 0, 0)),
            pl.BlockSpec((NCH, 1, CW), lambda i: (0, 0, 0)),
            pl.BlockSpec((D_H, D_H), lambda i: (0, 0)),
        ],
        out_specs=pl.BlockSpec((NCH, MBLK, CW), lambda i: (0, i, 0)),
        out_shape=jax.ShapeDtypeStruct((NCH, N, CW), jnp.float32),
    )(z1, y1, dis_b, b1c, gc, btc, W2)


def _fin_body(z_ref, y_ref, dis_ref, b2_ref, out_ref):
    dis = dis_ref[...]
    for ci in range(NCH):
        out_ref[:, ci * CW:(ci + 1) * CW] = (
            dis * (z_ref[ci] + y_ref[ci]) + b2_ref[ci])


def _fin_call(z2, y2, dis_b, b2c):
    return pl.pallas_call(
        _fin_body,
        grid=(GRID_M,),
        in_specs=[
            pl.BlockSpec((NCH, MBLK, CW), lambda i: (0, i, 0)),
            pl.BlockSpec((NCH, MBLK, CW), lambda i: (0, i, 0)),
            pl.BlockSpec((MBLK, CW), lambda i: (i, 0)),
            pl.BlockSpec((NCH, 1, CW), lambda i: (0, 0, 0)),
        ],
        out_specs=pl.BlockSpec((MBLK, D_H), lambda i: (i, 0)),
        out_shape=jax.ShapeDtypeStruct((N, D_H), jnp.float32),
    )(z2, y2, dis_b, b2c)


def kernel(x, edge_index, W1, b1, gamma, beta, W2, b2):
    src = edge_index[0].astype(jnp.int32)
    dst = edge_index[1].astype(jnp.int32)

    srcp = jnp.concatenate(
        [src, jnp.zeros((EP - E,), jnp.int32)]).reshape(NS, NBATCH, K)
    dstp = jnp.concatenate(
        [dst, jnp.full((EP - E,), N, jnp.int32)]).reshape(NS, NBATCH, K)
    dstd = jnp.concatenate(
        [dst, jnp.full((EPD - E,), N, jnp.int32)]).reshape(NC * NS, DEG_PER_W)

    b1c = b1.reshape(NCH, 1, CW)
    gc = gamma.reshape(NCH, 1, CW)
    btc = beta.reshape(NCH, 1, CW)
    b2c = b2.reshape(NCH, 1, CW)

    hist = _deg_call(dstd)
    dis_b = _dis_call(hist)
    y1 = _mm1_call(x, W1, dis_b)
    z1 = _scatter_call(y1, srcp, dstp)
    y2 = _mid_call(z1, y1, dis_b, b1c, gc, btc, W2)
    z2 = _scatter_call(y2, srcp, dstp)
    out = _fin_call(z2, y2, dis_b, b2c)
    return out


# speedup vs baseline: 6.5497x; 1.1585x over previous
import functools

import jax
import jax.numpy as jnp
from jax import lax
from jax.experimental import pallas as pl
from jax.experimental.pallas import tpu as pltpu
from jax.experimental.pallas import tpu_sc as plsc

N = 10000
E = 160000
D_IN = 256
D_H = 512
NCH = 4
CW = 128
NPAD = 10112
NB = 10240
NC, NS = 2, 16
NBATCH = 160
K = 64
NBUF = 4
DGRP = 8
NG = NBATCH // DGRP
EP = NS * NBATCH * K
DEG_PER_W = 5008
EPD = NC * NS * DEG_PER_W
MBLK = 400
GRID_M = N // MBLK
BN_SCALE = (1.0 + 1e-5) ** -0.5

_mesh = plsc.VectorSubcoreMesh(core_axis_name="c", subcore_axis_name="s")
_sc_params = pltpu.CompilerParams(needs_layout_passes=False)


def _deg_body(dst_hbm, hist_hbm, idx_v, hist_v):
    c = lax.axis_index("c")
    s = lax.axis_index("s")
    wid = c * NS + s
    pltpu.sync_copy(dst_hbm.at[wid], idx_v)
    zeros16 = jnp.zeros((16,), jnp.float32)
    ones16 = jnp.ones((16,), jnp.float32)

    def zbody(i, carry):
        hist_v[pl.ds(i * 16, 16)] = zeros16
        return carry

    lax.fori_loop(0, NB // 16, zbody, 0)

    def hbody(j, carry):
        idx = idx_v[pl.ds(j * 16, 16)]
        plsc.addupdate_scatter(hist_v, [idx], ones16)
        return carry

    lax.fori_loop(0, DEG_PER_W // 16, hbody, 0)
    pltpu.sync_copy(hist_v, hist_hbm.at[wid])


_deg_call = functools.partial(
    pl.kernel,
    out_type=jax.ShapeDtypeStruct((NC * NS, NB), jnp.float32),
    mesh=_mesh,
    scratch_types=[
        pltpu.VMEM((DEG_PER_W,), jnp.int32),
        pltpu.VMEM((NB,), jnp.float32),
    ],
    compiler_params=_sc_params,
)(_deg_body)


def _scatter_body(y_hbm, src_hbm, dst_hbm, z_hbm,
                  src_v, dbuf, rb0, rb1, rb2, rb3, acc,
                  gsem, ssem, isem):
    c = lax.axis_index("c")
    s = lax.axis_index("s")
    rowbufs = [rb0, rb1, rb2, rb3]
    pltpu.sync_copy(src_hbm.at[s], src_v)

    zeros16 = jnp.zeros((16,), jnp.float32)

    own = [(r0, 64) for r0 in range(0, 576, 64)] + [(576, 56)]
    base = s * (NPAD // NS)

    def process_chunk(chunk):
        y_view = y_hbm.at[chunk]
        z_view = z_hbm.at[chunk]

        def zb(r, carry):
            for k in range(CW // 16):
                rb0[r, pl.ds(k * 16, 16)] = zeros16
            return carry

        lax.fori_loop(0, K, zb, 0)
        for r0, sz in own:
            pltpu.sync_copy(rb0.at[pl.ds(0, sz)],
                            acc.at[pl.ds(base + r0, sz)])
        plsc.subcore_barrier()

        pltpu.sync_copy(dst_hbm.at[s, pl.ds(0, DGRP)], dbuf.at[0])
        pltpu.async_copy(y_view.at[src_v.at[0, pl.ds(0, K)]], rb0,
                         gsem.at[0])
        pltpu.async_copy(y_view.at[src_v.at[0, pl.ds(K, K)]], rb1,
                         gsem.at[1])

        def gbody(g, carry):
            p = lax.rem(g, 2)
            for b in range(DGRP):
                j = g * DGRP + b
                bf = b % NBUF
                rb = rowbufs[bf]

                @pl.when(j >= 2)
                def _():
                    wf = (b + 2) % NBUF
                    pltpu.make_async_copy(
                        rowbufs[wf], acc.at[dbuf.at[p, b]],
                        ssem.at[wf]).wait()

                @pl.when(j + 2 < NBATCH)
                def _():
                    nf = (b + 2) % NBUF
                    sidx = src_v.at[4 * g + (b + 2) // 2,
                                    pl.ds(((b + 2) % 2) * K, K)]
                    pltpu.async_copy(y_view.at[sidx], rowbufs[nf],
                                     gsem.at[nf])

                if b == 2:
                    @pl.when(g + 1 < NG)
                    def _():
                        pltpu.async_copy(
                            dst_hbm.at[s, pl.ds((g + 1) * DGRP, DGRP)],
                            dbuf.at[1 - p], isem)

                widx = src_v.at[4 * g + b // 2, pl.ds((b % 2) * K, K)]
                pltpu.make_async_copy(y_view.at[widx], rb,
                                      gsem.at[bf]).wait()
                pltpu.async_copy(rb, acc.at[dbuf.at[p, b]], ssem.at[bf],
                                 add=True)

            @pl.when(g + 1 < NG)
            def _():
                pltpu.make_async_copy(dst_hbm.at[s, pl.ds(0, DGRP)],
                                      dbuf.at[1 - p], isem).wait()
            return carry

        lax.fori_loop(0, NG, gbody, 0)
        for j in (NBATCH - 2, NBATCH - 1):
            bf = j % NBUF
            pltpu.make_async_copy(rowbufs[bf], acc.at[dbuf.at[0, 0]],
                                  ssem.at[bf]).wait()
        plsc.subcore_barrier()
        for r0, sz in own:
            pltpu.sync_copy(acc.at[pl.ds(base + r0, sz)],
                            z_view.at[pl.ds(base + r0, sz)])

    for cc in range(NCH // NC):
        for core in range(NC):
            @pl.when(c == core)
            def _(core=core, cc=cc):
                process_chunk(core * (NCH // NC) + cc)


_scatter_call = functools.partial(
    pl.kernel,
    out_type=jax.ShapeDtypeStruct((NCH, NPAD, CW), jnp.float32),
    mesh=_mesh,
    scratch_types=[
        pltpu.VMEM((NBATCH // 2, 2 * K), jnp.int32),
        pltpu.VMEM((2, DGRP, K), jnp.int32),
        pltpu.VMEM((K, CW), jnp.float32),
        pltpu.VMEM((K, CW), jnp.float32),
        pltpu.VMEM((K, CW), jnp.float32),
        pltpu.VMEM((K, CW), jnp.float32),
        pltpu.VMEM_SHARED((NPAD, CW), jnp.float32),
        pltpu.SemaphoreType.DMA((NBUF,)),
        pltpu.SemaphoreType.DMA((NBUF,)),
        pltpu.SemaphoreType.DMA,
    ],
    compiler_params=_sc_params,
)(_scatter_body)


def _dis_body(hist_ref, out_ref):
    h = hist_ref[...]
    deg = jnp.sum(h, axis=0, keepdims=True) + 1.0
    d = lax.rsqrt(deg)
    dcol = lax.transpose(d, (1, 0))
    out_ref[...] = jnp.broadcast_to(dcol[:N], (N, CW))


def _dis_call(hist):
    return pl.pallas_call(
        _dis_body,
        out_shape=jax.ShapeDtypeStruct((N, CW), jnp.float32),
    )(hist)


def _mm1_body(x_ref, w_ref, dis_ref, out_ref):
    xb = x_ref[...]
    w = w_ref[...]
    dis = dis_ref[...]
    for ci in range(NCH):
        out_ref[ci] = dis * jnp.dot(xb, w[:, ci * CW:(ci + 1) * CW],
                                    preferred_element_type=jnp.float32)


def _mm1_call(x, W1, dis_b):
    return pl.pallas_call(
        _mm1_body,
        grid=(GRID_M,),
        in_specs=[
            pl.BlockSpec((MBLK, D_IN), lambda i: (i, 0)),
            pl.BlockSpec((D_IN, D_H), lambda i: (0, 0)),
            pl.BlockSpec((MBLK, CW), lambda i: (i, 0)),
        ],
        out_specs=pl.BlockSpec((NCH, MBLK, CW), lambda i: (0, i, 0)),
        out_shape=jax.ShapeDtypeStruct((NCH, N, CW), jnp.float32),
    )(x, W1, dis_b)


def _mid_body(z_ref, y_ref, dis_ref, b1_ref, g_ref, bt_ref, w2_ref, out_ref):
    dis = dis_ref[...]
    w2 = w2_ref[...]
    acc = jnp.zeros((MBLK, D_H), jnp.float32)
    for ci in range(NCH):
        t = dis * (z_ref[ci] + y_ref[ci]) + b1_ref[ci]
        t = t * (BN_SCALE * g_ref[ci]) + bt_ref[ci]
        t = jnp.maximum(t, 0.0)
        acc = acc + jnp.dot(t, w2[ci * CW:(ci + 1) * CW, :],
                            preferred_element_type=jnp.float32)
    y2 = dis[:, 0:1] * acc
    for co in range(NCH):
        out_ref[co] = y2[:, co * CW:(co + 1) * CW]


def _mid_call(z1, y1, dis_b, b1c, gc, btc, W2):
    return pl.pallas_call(
        _mid_body,
        grid=(GRID_M,),
        in_specs=[
            pl.BlockSpec((NCH, MBLK, CW), lambda i: (0, i, 0)),
            pl.BlockSpec((NCH, MBLK, CW), lambda i: (0, i, 0)),
            pl.BlockSpec((MBLK, CW), lambda i: (i, 0)),
            pl.BlockSpec((NCH, 1, CW), lambda i: (0, 0, 0)),
            pl.BlockSpec((NCH, 1, CW), lambda i: (0, 0, 0)),
            pl.BlockSpec((NCH, 1, CW), lambda i: (0, 0, 0)),
            pl.BlockSpec((D_H, D_H), lambda i: (0, 0)),
        ],
        out_specs=pl.BlockSpec((NCH, MBLK, CW), lambda i: (0, i, 0)),
        out_shape=jax.ShapeDtypeStruct((NCH, N, CW), jnp.float32),
    )(z1, y1, dis_b, b1c, gc, btc, W2)


def _fin_body(z_ref, y_ref, dis_ref, b2_ref, out_ref):
    dis = dis_ref[...]
    for ci in range(NCH):
        out_ref[:, ci * CW:(ci + 1) * CW] = (
            dis * (z_ref[ci] + y_ref[ci]) + b2_ref[ci])


def _fin_call(z2, y2, dis_b, b2c):
    return pl.pallas_call(
        _fin_body,
        grid=(GRID_M,),
        in_specs=[
            pl.BlockSpec((NCH, MBLK, CW), lambda i: (0, i, 0)),
            pl.BlockSpec((NCH, MBLK, CW), lambda i: (0, i, 0)),
            pl.BlockSpec((MBLK, CW), lambda i: (i, 0)),
            pl.BlockSpec((NCH, 1, CW), lambda i: (0, 0, 0)),
        ],
        out_specs=pl.BlockSpec((MBLK, D_H), lambda i: (i, 0)),
        out_shape=jax.ShapeDtypeStruct((N, D_H), jnp.float32),
    )(z2, y2, dis_b, b2c)


def kernel(x, edge_index, W1, b1, gamma, beta, W2, b2):
    src = edge_index[0].astype(jnp.int32)
    dst = edge_index[1].astype(jnp.int32)

    srcp = jnp.concatenate(
        [src, jnp.zeros((EP - E,), jnp.int32)]).reshape(NS, NBATCH // 2,
                                                        2 * K)
    dstp = jnp.concatenate(
        [dst, jnp.full((EP - E,), N, jnp.int32)]).reshape(NS, NBATCH, K)
    dstd = jnp.concatenate(
        [dst, jnp.full((EPD - E,), N, jnp.int32)]).reshape(NC * NS, DEG_PER_W)

    b1c = b1.reshape(NCH, 1, CW)
    gc = gamma.reshape(NCH, 1, CW)
    btc = beta.reshape(NCH, 1, CW)
    b2c = b2.reshape(NCH, 1, CW)

    hist = _deg_call(dstd)
    dis_b = _dis_call(hist)
    y1 = _mm1_call(x, W1, dis_b)
    z1 = _scatter_call(y1, srcp, dstp)
    y2 = _mid_call(z1, y1, dis_b, b1c, gc, btc, W2)
    z2 = _scatter_call(y2, srcp, dstp)
    out = _fin_call(z2, y2, dis_b, b2c)
    return out
